# baseline (speedup 1.0000x reference)
import functools

import jax
import jax.numpy as jnp
from jax import lax
from jax.experimental import pallas as pl
from jax.experimental.pallas import tpu as pltpu
from jax.experimental.pallas import tpu_sc as plsc

_BATCH = 4096
_SEQ = 200
_EMBED = 64
_VOCAB = 100000
_NC = 2
_NS = 16
_NW = _NC * _NS
_ROWS_PER_W = _BATCH // _NW
_SC_CHUNK = 2
_TOK_CHUNK = _SC_CHUNK * _ROWS_PER_W
_NCHUNKS = _SEQ // _SC_CHUNK
_LANES = 16


def _sc_embed(tokens2d, token_table, pos_table):
    mesh = plsc.VectorSubcoreMesh(
        core_axis_name="c", subcore_axis_name="s",
        num_cores=_NC, num_subcores=_NS)

    out5 = jax.ShapeDtypeStruct(
        (_SEQ, _EMBED // 8, _BATCH // 128, 8, 128), jnp.float32)

    @functools.partial(
        pl.kernel,
        out_type=out5,
        mesh=mesh,
        compiler_params=pltpu.CompilerParams(use_tc_tiling_on_sc=False,
                                             needs_layout_passes=False),
        scratch_types=[
            pltpu.VMEM((_ROWS_PER_W, _SEQ + 1), jnp.int32),
            pltpu.VMEM((_SEQ, _EMBED), jnp.float32),
            [pltpu.VMEM((_TOK_CHUNK,), jnp.int32)] * 2,
            [pltpu.VMEM((_TOK_CHUNK, _EMBED), jnp.float32)] * 2,
            [pltpu.VMEM((_SC_CHUNK, _EMBED // 8, 8, 129), jnp.float32)] * 2,
            [pltpu.SemaphoreType.DMA] * 2,
            [pltpu.SemaphoreType.DMA] * 2,
            pltpu.SemaphoreType.DMA,
        ],
    )
    def k(tok_hbm, table_hbm, pos_hbm, out_hbm, toks_v, pos_v,
          idx_v, grows_v, trans_v, sem_g, sem_out, sem_in):
        cid = lax.axis_index("c")
        sid = lax.axis_index("s")
        wid = sid * _NC + cid

        tok_slab = tok_hbm.at[pl.ds(wid * _ROWS_PER_W, _ROWS_PER_W), :]
        toks_dst = toks_v.at[:, pl.ds(0, _SEQ)]
        pltpu.async_copy(tok_slab, toks_dst, sem_in)
        pltpu.async_copy(pos_hbm, pos_v, sem_in).wait()
        pltpu.make_async_copy(tok_slab, toks_dst, sem_in).wait()

        iota = lax.iota(jnp.int32, _LANES)

        def build_idx(g, p):
            for sl in range(_SC_CHUNK):
                s_vec = jnp.full((_LANES,), g * _SC_CHUNK + sl, jnp.int32)
                for v in range(_ROWS_PER_W // _LANES):
                    vals = plsc.load_gather(toks_v, [iota + v * _LANES, s_vec])
                    idx_v[p][pl.ds(sl * _ROWS_PER_W + v * _LANES, _LANES)] = vals

        def start_gather(p):
            pltpu.async_copy(table_hbm.at[idx_v[p]], grows_v[p], sem_g[p])

        nk = _EMBED // _LANES
        ehi_vecs = [(iota + k * _LANES) // 8 for k in range(nk)]
        elo_vecs = [(iota + k * _LANES) % 8 for k in range(nk)]

        def transpose_add(g, p):
            for sl in range(_SC_CHUNK):
                s_row = g * _SC_CHUNK + sl
                sl_vec = jnp.full((_LANES,), sl, jnp.int32)
                pos_vecs = [pos_v[s_row, pl.ds(k * _LANES, _LANES)]
                            for k in range(nk)]

                @plsc.parallel_loop(0, _ROWS_PER_W // _LANES, unroll=8)
                def vbody(v):
                    for r in range(_LANES):
                        b = v * _LANES + r
                        row = sl * _ROWS_PER_W + b
                        bsplat = jnp.full((_LANES,), b, jnp.int32)
                        for kk in range(nk):
                            vals = (grows_v[p][row, pl.ds(kk * _LANES, _LANES)]
                                    + pos_vecs[kk])
                            plsc.store_scatter(
                                trans_v[p],
                                [sl_vec, ehi_vecs[kk], elo_vecs[kk], bsplat],
                                vals)

        def out_copy(g, p):
            return pltpu.make_async_copy(
                trans_v[p].at[:, :, :, pl.ds(0, 128)],
                out_hbm.at[pl.ds(g * _SC_CHUNK, _SC_CHUNK), :, wid],
                sem_out[p])

        build_idx(0, 0)
        start_gather(0)

        def body(i, carry):
            for p in range(2):
                g = i * 2 + p
                q = 1 - p
                pltpu.make_async_copy(table_hbm.at[idx_v[p]], grows_v[p],
                                      sem_g[p]).wait()

                @pl.when(g + 1 < _NCHUNKS)
                def _():
                    build_idx(g + 1, q)
                    start_gather(q)

                @pl.when(g >= 2)
                def _():
                    out_copy(g - 2, p).wait()

                transpose_add(g, p)
                out_copy(g, p).start()
            return carry

        lax.fori_loop(0, _NCHUNKS // 2, body, 0)
        for p in range(2):
            out_copy(_NCHUNKS - 2 + p, p).wait()

    return k(tokens2d, token_table, pos_table)


def kernel(tokens, token_table, pos_table):
    out5 = _sc_embed(tokens.astype(jnp.int32), token_table, pos_table)
    return out5.transpose(2, 4, 0, 1, 3).reshape(_BATCH, _SEQ, _EMBED)


# speedup vs baseline: 1.4570x; 1.4570x over previous
import functools

import jax
import jax.numpy as jnp
from jax import lax
from jax.experimental import pallas as pl
from jax.experimental.pallas import tpu as pltpu
from jax.experimental.pallas import tpu_sc as plsc

_BATCH = 4096
_SEQ = 200
_EMBED = 64
_VOCAB = 100000
_NC = 2
_NS = 16
_NW = _NC * _NS
_ROWS_PER_W = _BATCH // _NW
_SC_CHUNK = 2
_TOK_CHUNK = _SC_CHUNK * _ROWS_PER_W
_NCHUNKS = _SEQ // _SC_CHUNK
_LANES = 16


def _sc_embed(tokens2d, token_table, pos_table):
    mesh = plsc.VectorSubcoreMesh(
        core_axis_name="c", subcore_axis_name="s",
        num_cores=_NC, num_subcores=_NS)

    out5 = jax.ShapeDtypeStruct(
        (_SEQ, _EMBED // 8, _BATCH // 128, 8, 128), jnp.float32)

    @functools.partial(
        pl.kernel,
        out_type=out5,
        mesh=mesh,
        compiler_params=pltpu.CompilerParams(use_tc_tiling_on_sc=False,
                                             needs_layout_passes=False),
        scratch_types=[
            pltpu.VMEM((_ROWS_PER_W, _SEQ + 1), jnp.int32),
            pltpu.VMEM((_SEQ, _EMBED), jnp.float32),
            [pltpu.VMEM((_TOK_CHUNK,), jnp.int32)] * 2,
            [pltpu.VMEM((_TOK_CHUNK, _EMBED), jnp.float32)] * 2,
            [pltpu.VMEM((_SC_CHUNK, _EMBED // 8, 8, 129), jnp.float32)] * 2,
            [pltpu.SemaphoreType.DMA] * 2,
            [pltpu.SemaphoreType.DMA] * 2,
            pltpu.SemaphoreType.DMA,
        ],
    )
    def k(tok_hbm, table_hbm, pos_hbm, out_hbm, toks_v, pos_v,
          idx_v, grows_v, trans_v, sem_g, sem_out, sem_in):
        cid = lax.axis_index("c")
        sid = lax.axis_index("s")
        wid = sid * _NC + cid

        tok_slab = tok_hbm.at[pl.ds(wid * _ROWS_PER_W, _ROWS_PER_W), :]
        toks_dst = toks_v.at[:, pl.ds(0, _SEQ)]
        pltpu.async_copy(tok_slab, toks_dst, sem_in)
        pltpu.async_copy(pos_hbm, pos_v, sem_in).wait()
        pltpu.make_async_copy(tok_slab, toks_dst, sem_in).wait()

        iota = lax.iota(jnp.int32, _LANES)

        def build_idx(g, p):
            for sl in range(_SC_CHUNK):
                s_vec = jnp.full((_LANES,), g * _SC_CHUNK + sl, jnp.int32)
                for v in range(_ROWS_PER_W // _LANES):
                    vals = plsc.load_gather(toks_v, [iota + v * _LANES, s_vec])
                    idx_v[p][pl.ds(sl * _ROWS_PER_W + v * _LANES, _LANES)] = vals

        def start_gather(p):
            pltpu.async_copy(table_hbm.at[idx_v[p]], grows_v[p], sem_g[p])

        nk = _EMBED // _LANES
        ehi_vecs = [(iota + k * _LANES) // 8 for k in range(nk)]
        elo_vecs = [(iota + k * _LANES) % 8 for k in range(nk)]

        def transpose_add(g, p):
            pos_all = [[pos_v[g * _SC_CHUNK + sl, pl.ds(k * _LANES, _LANES)]
                        for k in range(nk)] for sl in range(_SC_CHUNK)]
            sl_vecs = [jnp.full((_LANES,), sl, jnp.int32)
                       for sl in range(_SC_CHUNK)]
            nv = _ROWS_PER_W // _LANES

            @plsc.parallel_loop(0, _SC_CHUNK * nv, unroll=4)
            def vbody(u):
                v = u % nv
                sl_i = u // nv
                for sl in range(_SC_CHUNK):
                    @pl.when(sl_i == sl)
                    def _():
                        for r in range(_LANES):
                            b = v * _LANES + r
                            row = sl * _ROWS_PER_W + b
                            bsplat = jnp.full((_LANES,), b, jnp.int32)
                            for kk in range(nk):
                                vals = (grows_v[p][row,
                                                   pl.ds(kk * _LANES, _LANES)]
                                        + pos_all[sl][kk])
                                plsc.store_scatter(
                                    trans_v[p],
                                    [sl_vecs[sl], ehi_vecs[kk], elo_vecs[kk],
                                     bsplat],
                                    vals)

        def out_copy(g, p):
            return pltpu.make_async_copy(
                trans_v[p].at[:, :, :, pl.ds(0, 128)],
                out_hbm.at[pl.ds(g * _SC_CHUNK, _SC_CHUNK), :, wid],
                sem_out[p])

        build_idx(0, 0)
        start_gather(0)

        def body(i, carry):
            for p in range(2):
                g = i * 2 + p
                q = 1 - p
                pltpu.make_async_copy(table_hbm.at[idx_v[p]], grows_v[p],
                                      sem_g[p]).wait()

                @pl.when(g + 1 < _NCHUNKS)
                def _():
                    build_idx(g + 1, q)
                    start_gather(q)

                @pl.when(g >= 2)
                def _():
                    out_copy(g - 2, p).wait()

                transpose_add(g, p)
                out_copy(g, p).start()
            return carry

        lax.fori_loop(0, _NCHUNKS // 2, body, 0)
        for p in range(2):
            out_copy(_NCHUNKS - 2 + p, p).wait()

    return k(tokens2d, token_table, pos_table)


def kernel(tokens, token_table, pos_table):
    out5 = _sc_embed(tokens.astype(jnp.int32), token_table, pos_table)
    return out5.transpose(2, 4, 0, 1, 3).reshape(_BATCH, _SEQ, _EMBED)
